# Initial kernel scaffold; baseline (speedup 1.0000x reference)
import jax
import jax.numpy as jnp
from jax.experimental import pallas as pl


def kernel(h, memory, Wq, bq, Wo, bo):
    raise NotImplementedError("write your pallas kernel here")


# speedup vs baseline: 2.1022x; 2.1022x over previous
import jax
import jax.numpy as jnp
from jax.experimental import pallas as pl
from jax.experimental.pallas import tpu as pltpu

HIDDEN_DIM = 4096
MEMORY_DIM = 256
NUM_SLOTS = 65536
BLOCK_SLOTS = 4096
NUM_BLOCKS = NUM_SLOTS // BLOCK_SLOTS
TOPK = 5


def _flash_body(h_ref, mem_ref, wq_ref, bq_ref, wo_ref, bo_ref,
                out_ref, top_ref,
                q_scr, m_scr, l_scr, r_scr, sc_scr):
    i = pl.program_id(0)

    @pl.when(i == 0)
    def _init():
        q = jax.lax.dot_general(
            h_ref[...], wq_ref[...], (((1,), (1,)), ((), ())),
            preferred_element_type=jnp.float32)
        q_scr[...] = (q + bq_ref[...]) * (1.0 / 16.0)
        m_scr[0, 0] = -jnp.inf
        l_scr[0, 0] = 0.0
        r_scr[...] = jnp.zeros_like(r_scr)

    mem = mem_ref[...]
    s = jax.lax.dot_general(
        q_scr[...], mem, (((1,), (1,)), ((), ())),
        preferred_element_type=jnp.float32)
    sc_scr[:, pl.ds(i * BLOCK_SLOTS, BLOCK_SLOTS)] = s

    m_old = m_scr[0, 0]
    m_new = jnp.maximum(m_old, jnp.max(s))
    alpha = jnp.exp(m_old - m_new)
    p = jnp.exp(s - m_new)
    l_scr[0, 0] = l_scr[0, 0] * alpha + jnp.sum(p)
    r_blk = jax.lax.dot_general(
        p, mem, (((1,), (0,)), ((), ())),
        preferred_element_type=jnp.float32)
    r_scr[...] = r_scr[...] * alpha + r_blk
    m_scr[0, 0] = m_new

    @pl.when(i == NUM_BLOCKS - 1)
    def _finish():
        retrieved = r_scr[...] / l_scr[0, 0]
        out = jax.lax.dot_general(
            retrieved, wo_ref[...], (((1,), (1,)), ((), ())),
            preferred_element_type=jnp.float32)
        out_ref[...] = out + bo_ref[...]

        sc = sc_scr[...]
        idxs = jax.lax.broadcasted_iota(jnp.int32, sc.shape, 1)
        lane = jax.lax.broadcasted_iota(jnp.int32, (1, 128), 1)
        top = jnp.zeros((1, 128), jnp.int32)
        for k in range(TOPK):
            mv = jnp.max(sc)
            t = jnp.min(jnp.where(sc == mv, idxs, NUM_SLOTS))
            top = jnp.where(lane == k, t, top)
            sc = jnp.where(idxs == t, -jnp.inf, sc)
        top_ref[...] = top


def kernel(h, memory, Wq, bq, Wo, bo):
    h2 = h.reshape(1, HIDDEN_DIM)
    bq2 = bq.reshape(1, MEMORY_DIM)
    bo2 = bo.reshape(1, HIDDEN_DIM)
    out, top = pl.pallas_call(
        _flash_body,
        grid=(NUM_BLOCKS,),
        in_specs=[
            pl.BlockSpec((1, HIDDEN_DIM), lambda i: (0, 0)),
            pl.BlockSpec((BLOCK_SLOTS, MEMORY_DIM), lambda i: (i, 0)),
            pl.BlockSpec((MEMORY_DIM, HIDDEN_DIM), lambda i: (0, 0)),
            pl.BlockSpec((1, MEMORY_DIM), lambda i: (0, 0)),
            pl.BlockSpec((HIDDEN_DIM, MEMORY_DIM), lambda i: (0, 0)),
            pl.BlockSpec((1, HIDDEN_DIM), lambda i: (0, 0)),
        ],
        out_specs=[
            pl.BlockSpec((1, HIDDEN_DIM), lambda i: (0, 0)),
            pl.BlockSpec((1, 128), lambda i: (0, 0)),
        ],
        out_shape=[
            jax.ShapeDtypeStruct((1, HIDDEN_DIM), jnp.float32),
            jax.ShapeDtypeStruct((1, 128), jnp.int32),
        ],
        scratch_shapes=[
            pltpu.VMEM((1, MEMORY_DIM), jnp.float32),
            pltpu.SMEM((1, 1), jnp.float32),
            pltpu.SMEM((1, 1), jnp.float32),
            pltpu.VMEM((1, MEMORY_DIM), jnp.float32),
            pltpu.VMEM((1, NUM_SLOTS), jnp.float32),
        ],
        compiler_params=pltpu.CompilerParams(
            dimension_semantics=("arbitrary",)),
    )(h2, memory, Wq, bq2, Wo, bo2)
    return out.reshape(1, 1, HIDDEN_DIM), top[0, :TOPK]


# speedup vs baseline: 2.5828x; 1.2286x over previous
import jax
import jax.numpy as jnp
from jax.experimental import pallas as pl
from jax.experimental.pallas import tpu as pltpu

HIDDEN_DIM = 4096
MEMORY_DIM = 256
NUM_SLOTS = 65536
ROWS = 8
COLS = NUM_SLOTS // ROWS
BLOCK_COLS = 512
NUM_BLOCKS = COLS // BLOCK_COLS
TOPK = 5


def _flash_body(h_ref, mem_ref, wq_ref, bq_ref, wo_ref, bo_ref,
                out_ref, top_ref,
                q_scr, m_scr, l_scr, r_scr, sc_scr):
    i = pl.program_id(0)

    @pl.when(i == 0)
    def _init():
        q = jax.lax.dot_general(
            h_ref[...], wq_ref[...], (((1,), (1,)), ((), ())),
            preferred_element_type=jnp.float32)
        q_scr[...] = jnp.broadcast_to(
            (q + bq_ref[...]) * (1.0 / 16.0), (ROWS, MEMORY_DIM))
        m_scr[0, 0] = -jnp.inf
        l_scr[0, 0] = 0.0
        r_scr[...] = jnp.zeros_like(r_scr)

    mem = mem_ref[...]
    s = jax.lax.dot_general(
        q_scr[...], mem, (((1,), (2,)), ((0,), (0,))),
        preferred_element_type=jnp.float32)
    sc_scr[:, pl.ds(i * BLOCK_COLS, BLOCK_COLS)] = s

    m_old = m_scr[0, 0]
    m_new = jnp.maximum(m_old, jnp.max(s))
    alpha = jnp.exp(m_old - m_new)
    p = jnp.exp(s - m_new)
    l_scr[0, 0] = l_scr[0, 0] * alpha + jnp.sum(p)
    r_blk = jax.lax.dot_general(
        p, mem, (((1,), (1,)), ((0,), (0,))),
        preferred_element_type=jnp.float32)
    r_scr[...] = r_scr[...] * alpha + r_blk
    m_scr[0, 0] = m_new

    @pl.when(i == NUM_BLOCKS - 1)
    def _finish():
        retrieved = jnp.sum(r_scr[...], axis=0, keepdims=True) / l_scr[0, 0]
        out = jax.lax.dot_general(
            retrieved, wo_ref[...], (((1,), (1,)), ((), ())),
            preferred_element_type=jnp.float32)
        out_ref[...] = out + bo_ref[...]

        sc = sc_scr[...]
        idxs = (jax.lax.broadcasted_iota(jnp.int32, sc.shape, 0) * COLS
                + jax.lax.broadcasted_iota(jnp.int32, sc.shape, 1))
        lane = jax.lax.broadcasted_iota(jnp.int32, (1, 128), 1)
        top = jnp.zeros((1, 128), jnp.int32)
        for k in range(TOPK):
            mv = jnp.max(sc)
            t = jnp.min(jnp.where(sc == mv, idxs, NUM_SLOTS))
            top = jnp.where(lane == k, t, top)
            sc = jnp.where(idxs == t, -jnp.inf, sc)
        top_ref[...] = top


def kernel(h, memory, Wq, bq, Wo, bo):
    h2 = h.reshape(1, HIDDEN_DIM)
    bq2 = bq.reshape(1, MEMORY_DIM)
    bo2 = bo.reshape(1, HIDDEN_DIM)
    mem3 = memory.reshape(ROWS, COLS, MEMORY_DIM)
    out, top = pl.pallas_call(
        _flash_body,
        grid=(NUM_BLOCKS,),
        in_specs=[
            pl.BlockSpec((1, HIDDEN_DIM), lambda i: (0, 0)),
            pl.BlockSpec((ROWS, BLOCK_COLS, MEMORY_DIM), lambda i: (0, i, 0)),
            pl.BlockSpec((MEMORY_DIM, HIDDEN_DIM), lambda i: (0, 0)),
            pl.BlockSpec((1, MEMORY_DIM), lambda i: (0, 0)),
            pl.BlockSpec((HIDDEN_DIM, MEMORY_DIM), lambda i: (0, 0)),
            pl.BlockSpec((1, HIDDEN_DIM), lambda i: (0, 0)),
        ],
        out_specs=[
            pl.BlockSpec((1, HIDDEN_DIM), lambda i: (0, 0)),
            pl.BlockSpec((1, 128), lambda i: (0, 0)),
        ],
        out_shape=[
            jax.ShapeDtypeStruct((1, HIDDEN_DIM), jnp.float32),
            jax.ShapeDtypeStruct((1, 128), jnp.int32),
        ],
        scratch_shapes=[
            pltpu.VMEM((ROWS, MEMORY_DIM), jnp.float32),
            pltpu.SMEM((1, 1), jnp.float32),
            pltpu.SMEM((1, 1), jnp.float32),
            pltpu.VMEM((ROWS, MEMORY_DIM), jnp.float32),
            pltpu.VMEM((ROWS, COLS), jnp.float32),
        ],
        compiler_params=pltpu.CompilerParams(
            dimension_semantics=("arbitrary",)),
    )(h2, mem3, Wq, bq2, Wo, bo2)
    return out.reshape(1, 1, HIDDEN_DIM), top[0, :TOPK]


# speedup vs baseline: 2.9464x; 1.1408x over previous
import jax
import jax.numpy as jnp
from jax.experimental import pallas as pl
from jax.experimental.pallas import tpu as pltpu

HIDDEN_DIM = 4096
MEMORY_DIM = 256
NUM_SLOTS = 65536
ROWS = 8
COLS = NUM_SLOTS // ROWS
BLOCK_COLS = 1024
NUM_BLOCKS = COLS // BLOCK_COLS
TOPK = 5


def _flash_body(h_ref, mem_ref, wq_ref, bq_ref, wo_ref, bo_ref,
                out_ref, top_ref,
                q_scr, m_scr, l_scr, r_scr, sc_scr):
    i = pl.program_id(0)

    @pl.when(i == 0)
    def _init():
        q = jax.lax.dot_general(
            h_ref[...], wq_ref[...], (((1,), (1,)), ((), ())),
            preferred_element_type=jnp.float32)
        q_scr[...] = jnp.broadcast_to(
            (q + bq_ref[...]) * (1.0 / 16.0), (ROWS, MEMORY_DIM))
        m_scr[0, 0] = -jnp.inf
        l_scr[0, 0] = 0.0
        r_scr[...] = jnp.zeros_like(r_scr)

    mem = mem_ref[...]
    s = jax.lax.dot_general(
        q_scr[...], mem, (((1,), (2,)), ((0,), (0,))),
        preferred_element_type=jnp.float32)
    sc_scr[:, pl.ds(i * BLOCK_COLS, BLOCK_COLS)] = s

    m_old = m_scr[0, 0]
    m_new = jnp.maximum(m_old, jnp.max(s))
    alpha = jnp.exp(m_old - m_new)
    p = jnp.exp(s - m_new)
    l_scr[0, 0] = l_scr[0, 0] * alpha + jnp.sum(p)
    r_blk = jax.lax.dot_general(
        p, mem, (((1,), (1,)), ((0,), (0,))),
        preferred_element_type=jnp.float32)
    r_scr[...] = r_scr[...] * alpha + r_blk
    m_scr[0, 0] = m_new

    @pl.when(i == NUM_BLOCKS - 1)
    def _finish():
        retrieved = jnp.sum(r_scr[...], axis=0, keepdims=True) / l_scr[0, 0]
        out = jax.lax.dot_general(
            retrieved, wo_ref[...], (((1,), (1,)), ((), ())),
            preferred_element_type=jnp.float32)
        out_ref[...] = out + bo_ref[...]

        sc = sc_scr[...]
        idxs = (jax.lax.broadcasted_iota(jnp.int32, sc.shape, 0) * COLS
                + jax.lax.broadcasted_iota(jnp.int32, sc.shape, 1))
        lane = jax.lax.broadcasted_iota(jnp.int32, (1, 128), 1)
        top = jnp.zeros((1, 128), jnp.int32)
        for k in range(TOPK):
            mv = jnp.max(sc)
            t = jnp.min(jnp.where(sc == mv, idxs, NUM_SLOTS))
            top = jnp.where(lane == k, t, top)
            sc = jnp.where(idxs == t, -jnp.inf, sc)
        top_ref[...] = top


def kernel(h, memory, Wq, bq, Wo, bo):
    h2 = h.reshape(1, HIDDEN_DIM)
    bq2 = bq.reshape(1, MEMORY_DIM)
    bo2 = bo.reshape(1, HIDDEN_DIM)
    mem3 = memory.reshape(ROWS, COLS, MEMORY_DIM)
    out, top = pl.pallas_call(
        _flash_body,
        grid=(NUM_BLOCKS,),
        in_specs=[
            pl.BlockSpec((1, HIDDEN_DIM), lambda i: (0, 0)),
            pl.BlockSpec((ROWS, BLOCK_COLS, MEMORY_DIM), lambda i: (0, i, 0)),
            pl.BlockSpec((MEMORY_DIM, HIDDEN_DIM), lambda i: (0, 0)),
            pl.BlockSpec((1, MEMORY_DIM), lambda i: (0, 0)),
            pl.BlockSpec((HIDDEN_DIM, MEMORY_DIM), lambda i: (0, 0)),
            pl.BlockSpec((1, HIDDEN_DIM), lambda i: (0, 0)),
        ],
        out_specs=[
            pl.BlockSpec((1, HIDDEN_DIM), lambda i: (0, 0)),
            pl.BlockSpec((1, 128), lambda i: (0, 0)),
        ],
        out_shape=[
            jax.ShapeDtypeStruct((1, HIDDEN_DIM), jnp.float32),
            jax.ShapeDtypeStruct((1, 128), jnp.int32),
        ],
        scratch_shapes=[
            pltpu.VMEM((ROWS, MEMORY_DIM), jnp.float32),
            pltpu.SMEM((1, 1), jnp.float32),
            pltpu.SMEM((1, 1), jnp.float32),
            pltpu.VMEM((ROWS, MEMORY_DIM), jnp.float32),
            pltpu.VMEM((ROWS, COLS), jnp.float32),
        ],
        compiler_params=pltpu.CompilerParams(
            dimension_semantics=("arbitrary",)),
    )(h2, mem3, Wq, bq2, Wo, bo2)
    return out.reshape(1, 1, HIDDEN_DIM), top[0, :TOPK]


# speedup vs baseline: 3.0002x; 1.0183x over previous
import jax
import jax.numpy as jnp
from jax.experimental import pallas as pl
from jax.experimental.pallas import tpu as pltpu

HIDDEN_DIM = 4096
MEMORY_DIM = 256
NUM_SLOTS = 65536
ROWS = 8
COLS = NUM_SLOTS // ROWS
BLOCK_COLS = 2048
NUM_BLOCKS = COLS // BLOCK_COLS
TOPK = 5


def _flash_body(h_ref, mem_ref, wq_ref, bq_ref, wo_ref, bo_ref,
                out_ref, top_ref,
                q_scr, m_scr, l_scr, r_scr, sc_scr):
    i = pl.program_id(0)

    @pl.when(i == 0)
    def _init():
        q = jax.lax.dot_general(
            h_ref[...], wq_ref[...], (((1,), (1,)), ((), ())),
            preferred_element_type=jnp.float32)
        q_scr[...] = jnp.broadcast_to(
            (q + bq_ref[...]) * (1.0 / 16.0), (ROWS, MEMORY_DIM))
        m_scr[0, 0] = -jnp.inf
        l_scr[0, 0] = 0.0
        r_scr[...] = jnp.zeros_like(r_scr)

    mem = mem_ref[...]
    s = jax.lax.dot_general(
        q_scr[...], mem, (((1,), (2,)), ((0,), (0,))),
        preferred_element_type=jnp.float32)
    sc_scr[:, pl.ds(i * BLOCK_COLS, BLOCK_COLS)] = s

    m_old = m_scr[0, 0]
    m_new = jnp.maximum(m_old, jnp.max(s))
    alpha = jnp.exp(m_old - m_new)
    p = jnp.exp(s - m_new)
    l_scr[0, 0] = l_scr[0, 0] * alpha + jnp.sum(p)
    r_blk = jax.lax.dot_general(
        p, mem, (((1,), (1,)), ((0,), (0,))),
        preferred_element_type=jnp.float32)
    r_scr[...] = r_scr[...] * alpha + r_blk
    m_scr[0, 0] = m_new

    @pl.when(i == NUM_BLOCKS - 1)
    def _finish():
        retrieved = jnp.sum(r_scr[...], axis=0, keepdims=True) / l_scr[0, 0]
        out = jax.lax.dot_general(
            retrieved, wo_ref[...], (((1,), (1,)), ((), ())),
            preferred_element_type=jnp.float32)
        out_ref[...] = out + bo_ref[...]

        sc = sc_scr[...]
        idxs = (jax.lax.broadcasted_iota(jnp.int32, sc.shape, 0) * COLS
                + jax.lax.broadcasted_iota(jnp.int32, sc.shape, 1))
        lane = jax.lax.broadcasted_iota(jnp.int32, (1, 128), 1)
        top = jnp.zeros((1, 128), jnp.int32)
        for k in range(TOPK):
            mv = jnp.max(sc)
            t = jnp.min(jnp.where(sc == mv, idxs, NUM_SLOTS))
            top = jnp.where(lane == k, t, top)
            sc = jnp.where(idxs == t, -jnp.inf, sc)
        top_ref[...] = top


def kernel(h, memory, Wq, bq, Wo, bo):
    h2 = h.reshape(1, HIDDEN_DIM)
    bq2 = bq.reshape(1, MEMORY_DIM)
    bo2 = bo.reshape(1, HIDDEN_DIM)
    mem3 = memory.reshape(ROWS, COLS, MEMORY_DIM)
    out, top = pl.pallas_call(
        _flash_body,
        grid=(NUM_BLOCKS,),
        in_specs=[
            pl.BlockSpec((1, HIDDEN_DIM), lambda i: (0, 0)),
            pl.BlockSpec((ROWS, BLOCK_COLS, MEMORY_DIM), lambda i: (0, i, 0)),
            pl.BlockSpec((MEMORY_DIM, HIDDEN_DIM), lambda i: (0, 0)),
            pl.BlockSpec((1, MEMORY_DIM), lambda i: (0, 0)),
            pl.BlockSpec((HIDDEN_DIM, MEMORY_DIM), lambda i: (0, 0)),
            pl.BlockSpec((1, HIDDEN_DIM), lambda i: (0, 0)),
        ],
        out_specs=[
            pl.BlockSpec((1, HIDDEN_DIM), lambda i: (0, 0)),
            pl.BlockSpec((1, 128), lambda i: (0, 0)),
        ],
        out_shape=[
            jax.ShapeDtypeStruct((1, HIDDEN_DIM), jnp.float32),
            jax.ShapeDtypeStruct((1, 128), jnp.int32),
        ],
        scratch_shapes=[
            pltpu.VMEM((ROWS, MEMORY_DIM), jnp.float32),
            pltpu.SMEM((1, 1), jnp.float32),
            pltpu.SMEM((1, 1), jnp.float32),
            pltpu.VMEM((ROWS, MEMORY_DIM), jnp.float32),
            pltpu.VMEM((ROWS, COLS), jnp.float32),
        ],
        compiler_params=pltpu.CompilerParams(
            dimension_semantics=("arbitrary",)),
    )(h2, mem3, Wq, bq2, Wo, bo2)
    return out.reshape(1, 1, HIDDEN_DIM), top[0, :TOPK]
